# Initial kernel scaffold; baseline (speedup 1.0000x reference)
import jax
import jax.numpy as jnp
from jax.experimental import pallas as pl


def kernel(x, edge_index, batch, params):
    raise NotImplementedError("write your pallas kernel here")


# speedup vs baseline: 1.0006x; 1.0006x over previous
import functools

import jax
import jax.numpy as jnp
from jax.experimental import pallas as pl
from jax.experimental.pallas import tpu as pltpu

N = 10000
E = 320000
G = 8
IN_DIM = 128
HID = 256
OUT = 256

_ROWS = 400


def _pool_body(x_ref, b_ref, sum_ref, cnt_ref):
    i = pl.program_id(0)

    @pl.when(i == 0)
    def _init():
        sum_ref[...] = jnp.zeros_like(sum_ref)
        cnt_ref[...] = jnp.zeros_like(cnt_ref)

    x = x_ref[...]
    b = b_ref[0, 0, :]
    gids = jax.lax.broadcasted_iota(jnp.int32, (G, _ROWS), 0)
    onehot = (b[None, :] == gids).astype(jnp.float32)
    sum_ref[...] += jnp.dot(onehot, x, preferred_element_type=jnp.float32)
    cnt_ref[...] += jnp.sum(onehot, axis=1, keepdims=True)


def _global_mean_pool(combined, batch):
    n = combined.shape[0]
    grid = n // _ROWS
    sums, cnt = pl.pallas_call(
        _pool_body,
        grid=(grid,),
        in_specs=[
            pl.BlockSpec((_ROWS, combined.shape[1]), lambda i: (i, 0)),
            pl.BlockSpec((1, 1, _ROWS), lambda i: (i, 0, 0)),
        ],
        out_specs=[
            pl.BlockSpec((G, combined.shape[1]), lambda i: (0, 0)),
            pl.BlockSpec((G, 1), lambda i: (0, 0)),
        ],
        out_shape=[
            jax.ShapeDtypeStruct((G, combined.shape[1]), jnp.float32),
            jax.ShapeDtypeStruct((G, 1), jnp.float32),
        ],
    )(combined, batch.reshape(grid, 1, _ROWS))
    return sums / jnp.maximum(cnt, 1.0)


def _seg_softmax(logits, seg, n):
    m = jax.ops.segment_max(logits, seg, num_segments=n)
    m = jnp.where(jnp.isfinite(m), m, 0.0)
    e = jnp.exp(logits - m[seg])
    s = jax.ops.segment_sum(e, seg, num_segments=n)
    return e / (s[seg] + 1e-16)


def _gcn_conv(x, s, d, W, b, norm):
    h = x @ W
    out = jax.ops.segment_sum(h[s] * norm[:, None], d, num_segments=x.shape[0])
    return out + b


def _batch_norm(x, gamma, beta, eps=1e-5):
    mu = x.mean(0)
    var = x.var(0)
    return (x - mu) * jax.lax.rsqrt(var + eps) * gamma + beta


def _gat_conv(x, s, d, W, a_src, a_dst, b, heads, out_ch, concat):
    n = x.shape[0]
    h = (x @ W).reshape(n, heads, out_ch)
    al_s = (h * a_src[None]).sum(-1)
    al_d = (h * a_dst[None]).sum(-1)
    e = jax.nn.leaky_relu(al_s[s] + al_d[d], 0.2)
    attn = _seg_softmax(e, d, n)
    out = jax.ops.segment_sum(h[s] * attn[:, :, None], d, num_segments=n)
    if concat:
        out = out.reshape(n, heads * out_ch)
    else:
        out = out.mean(1)
    return out + b


def _transformer_conv(x, src, dst, Wq, Wk, Wv, Ws, b, heads):
    n = x.shape[0]
    dh = Wq.shape[1] // heads
    q = (x @ Wq).reshape(n, heads, dh)
    k = (x @ Wk).reshape(n, heads, dh)
    v = (x @ Wv).reshape(n, heads, dh)
    logits = (q[dst] * k[src]).sum(-1) / jnp.sqrt(float(dh))
    attn = _seg_softmax(logits, dst, n)
    out = jax.ops.segment_sum(v[src] * attn[:, :, None], dst, num_segments=n)
    out = out.reshape(n, heads * dh)
    return out + x @ Ws + b


def kernel(x, edge_index, batch, params):
    p = params
    src, dst = edge_index[0], edge_index[1]
    n = x.shape[0]
    loop = jnp.arange(n, dtype=src.dtype)
    s = jnp.concatenate([src, loop])
    d = jnp.concatenate([dst, loop])
    x = x.astype(jnp.float32)

    ones = jnp.ones(s.shape[0], dtype=jnp.float32)
    deg = jax.ops.segment_sum(ones, d, num_segments=n)
    dinv = jnp.where(deg > 0, jax.lax.rsqrt(jnp.maximum(deg, 1e-12)), 0.0)
    norm = dinv[s] * dinv[d]

    gcn_out = x
    for i in range(4):
        W = p['gcn_W%d' % i]
        residual = gcn_out if gcn_out.shape[-1] == W.shape[1] else None
        gcn_out = _gcn_conv(gcn_out, s, d, W, p['gcn_b%d' % i], norm)
        gcn_out = _batch_norm(gcn_out, p['bn_g%d' % i], p['bn_b%d' % i])
        gcn_out = jax.nn.relu(gcn_out)
        if residual is not None and i > 0:
            gcn_out = gcn_out + residual

    gat_out = x
    cfgs = [(8, HID // 8, True), (4, HID // 4, True), (1, OUT, False)]
    for i, (hds, oc, cc) in enumerate(cfgs):
        gat_out = jax.nn.relu(
            _gat_conv(gat_out, s, d, p['gat%d_W' % i], p['gat%d_as' % i],
                      p['gat%d_ad' % i], p['gat%d_b' % i], hds, oc, cc))

    trans_out = _transformer_conv(gcn_out, src, dst, p['tr_Wq'], p['tr_Wk'],
                                  p['tr_Wv'], p['tr_Ws'], p['tr_b'], 8)
    combined = gcn_out + gat_out + trans_out
    return _global_mean_pool(combined, batch)


# speedup vs baseline: 10.1871x; 10.1806x over previous
import jax
import jax.numpy as jnp
from jax import lax
from jax.experimental import pallas as pl
from jax.experimental.pallas import tpu as pltpu
from jax.experimental.pallas import tpu_sc as plsc

N = 10000
E = 320000
G = 8
IN_DIM = 128
HID = 256
OUT = 256

NPAD = 10240
NC, NS = 2, 16
NW = NC * NS
EB = 128
EUP = E + N
EP = ((EUP + NW * EB - 1) // (NW * EB)) * (NW * EB)
EPT = ((E + NW * EB - 1) // (NW * EB)) * (NW * EB)
CH = 128

_mesh = plsc.VectorSubcoreMesh(
    core_axis_name="c", subcore_axis_name="s", num_cores=NC, num_subcores=NS)


def _sc_gather(table, idx, D):
    M = idx.shape[0]
    perw = M // NW
    nb = perw // EB

    def body(table_hbm, idx_hbm, out_hbm, idx_v, rows_v, sem):
        wid = lax.axis_index("s") * NC + lax.axis_index("c")
        base = wid * perw

        def step(j, c):
            off = base + j * EB
            pltpu.sync_copy(idx_hbm.at[pl.ds(off, EB)], idx_v)
            pltpu.async_copy(table_hbm.at[idx_v], rows_v, sem).wait()
            pltpu.sync_copy(rows_v, out_hbm.at[pl.ds(off, EB)])
            return c

        lax.fori_loop(0, nb, step, 0)

    f = pl.kernel(
        body,
        out_type=jax.ShapeDtypeStruct((M, D), jnp.float32),
        mesh=_mesh,
        scratch_types=[
            pltpu.VMEM((EB,), jnp.int32),
            pltpu.VMEM((EB, D), jnp.float32),
            pltpu.SemaphoreType.DMA,
        ],
    )
    return f(table, idx)


def _sc_segment_sum(parts, idx):
    M = idx.shape[0]
    perw = M // NW
    nb = perw // EB
    chunks = []
    for pi, a in enumerate(parts):
        for off in range(0, a.shape[1], CH):
            chunks.append((pi, off))
    Dtot = CH * len(chunks)
    RS = NPAD // NS

    def body(*refs):
        part_hbm = refs[:len(parts)]
        idx_hbm, out_hbm, idx_v, buf, acc = refs[len(parts):]
        cid = lax.axis_index("c")
        sid = lax.axis_index("s")
        base = (sid * NC + cid) * perw
        zero16 = jnp.zeros((16,), jnp.float32)

        for ci, (pi, off) in enumerate(chunks):
            def zrow(r, c):
                for jj in range(CH // 16):
                    buf[r, pl.ds(jj * 16, 16)] = zero16
                return c

            lax.fori_loop(0, EB, zrow, 0)
            for t in range(RS // EB):
                pltpu.sync_copy(buf, acc.at[pl.ds(sid * RS + t * EB, EB)])
            plsc.subcore_barrier()

            def step(j, c):
                eoff = base + j * EB
                pltpu.sync_copy(idx_hbm.at[pl.ds(eoff, EB)], idx_v)
                pltpu.sync_copy(
                    part_hbm[pi].at[pl.ds(eoff, EB), pl.ds(off, CH)], buf)
                pltpu.sync_copy(buf, acc.at[idx_v], add=True)
                return c

            lax.fori_loop(0, nb, step, 0)
            plsc.subcore_barrier()
            for t in range(RS // EB):
                r0 = sid * RS + t * EB
                pltpu.sync_copy(
                    acc.at[pl.ds(r0, EB)],
                    out_hbm.at[cid, pl.ds(r0, EB), pl.ds(ci * CH, CH)])
            plsc.subcore_barrier()

    f = pl.kernel(
        body,
        out_type=jax.ShapeDtypeStruct((NC, NPAD, Dtot), jnp.float32),
        mesh=_mesh,
        scratch_types=[
            pltpu.VMEM((EB,), jnp.int32),
            pltpu.VMEM((EB, CH), jnp.float32),
            pltpu.VMEM_SHARED((NPAD, CH), jnp.float32),
        ],
    )
    return f(*parts, idx)


def _sc_degree(idx):
    M = idx.shape[0]
    perw = M // NW
    nb = perw // EB
    RS = NPAD // NS

    def body(idx_hbm, out_hbm, idx_v, buf, acc):
        cid = lax.axis_index("c")
        sid = lax.axis_index("s")
        base = (sid * NC + cid) * perw
        zero16 = jnp.zeros((16,), jnp.float32)

        def zrow(r, c):
            for jj in range(CH // 16):
                buf[r, pl.ds(jj * 16, 16)] = zero16
            return c

        lax.fori_loop(0, EB, zrow, 0)
        for t in range(RS // EB):
            pltpu.sync_copy(buf, acc.at[pl.ds(sid * RS + t * EB, EB)])
        plsc.subcore_barrier()

        one16 = jnp.ones((16,), jnp.float32)

        def orow(r, c):
            for jj in range(CH // 16):
                buf[r, pl.ds(jj * 16, 16)] = one16
            return c

        lax.fori_loop(0, EB, orow, 0)

        def step(j, c):
            eoff = base + j * EB
            pltpu.sync_copy(idx_hbm.at[pl.ds(eoff, EB)], idx_v)
            pltpu.sync_copy(buf, acc.at[idx_v], add=True)
            return c

        lax.fori_loop(0, nb, step, 0)
        plsc.subcore_barrier()
        for t in range(RS // EB):
            r0 = sid * RS + t * EB
            pltpu.sync_copy(acc.at[pl.ds(r0, EB)],
                            out_hbm.at[cid, pl.ds(r0, EB)])
        plsc.subcore_barrier()

    f = pl.kernel(
        body,
        out_type=jax.ShapeDtypeStruct((NC, NPAD, CH), jnp.float32),
        mesh=_mesh,
        scratch_types=[
            pltpu.VMEM((EB,), jnp.int32),
            pltpu.VMEM((EB, CH), jnp.float32),
            pltpu.VMEM_SHARED((NPAD, CH), jnp.float32),
        ],
    )
    return f(idx)


_BM = 512


def _mm_body(x_ref, w_ref, b_ref, o_ref):
    o_ref[...] = jnp.dot(
        x_ref[...], w_ref[...], preferred_element_type=jnp.float32
    ) + b_ref[...]


def _tc_matmul(x, W, bias):
    M, K = x.shape
    Dout = W.shape[1]
    return pl.pallas_call(
        _mm_body,
        grid=(M // _BM,),
        in_specs=[
            pl.BlockSpec((_BM, K), lambda i: (i, 0)),
            pl.BlockSpec((K, Dout), lambda i: (0, 0)),
            pl.BlockSpec((1, Dout), lambda i: (0, 0)),
        ],
        out_specs=pl.BlockSpec((_BM, Dout), lambda i: (i, 0)),
        out_shape=jax.ShapeDtypeStruct((M, Dout), jnp.float32),
    )(x, W, bias.reshape(1, Dout))


def _dinv_body(h_ref, o_ref):
    d = h_ref[0] + h_ref[1]
    o_ref[...] = jnp.where(d > 0, lax.rsqrt(jnp.maximum(d, 1e-12)), 0.0)


def _tc_dinv(halves):
    return pl.pallas_call(
        _dinv_body,
        grid=(NPAD // _BM,),
        in_specs=[pl.BlockSpec((2, _BM, CH), lambda i: (0, i, 0))],
        out_specs=pl.BlockSpec((_BM, CH), lambda i: (i, 0)),
        out_shape=jax.ShapeDtypeStruct((NPAD, CH), jnp.float32),
    )(halves)


def _make_mul_body(mreal):
    def body(a_ref, b_ref, o_ref):
        i = pl.program_id(0)
        row = i * _BM + lax.broadcasted_iota(jnp.int32, (_BM, CH), 0)
        o_ref[...] = jnp.where(row < mreal, a_ref[...] * b_ref[...], 0.0)
    return body


def _tc_mul(a, b, mreal):
    M, C = a.shape
    return pl.pallas_call(
        _make_mul_body(mreal),
        grid=(M // _BM,),
        in_specs=[
            pl.BlockSpec((_BM, C), lambda i: (i, 0)),
            pl.BlockSpec((_BM, C), lambda i: (i, 0)),
        ],
        out_specs=pl.BlockSpec((_BM, C), lambda i: (i, 0)),
        out_shape=jax.ShapeDtypeStruct((M, C), jnp.float32),
    )(a, b)


def _make_gat_exp_body(mreal):
    def body(a_ref, b_ref, o_ref):
        i = pl.program_id(0)
        row = i * _BM + lax.broadcasted_iota(jnp.int32, (_BM, CH), 0)
        s = a_ref[...] + b_ref[...]
        e = jnp.exp(jnp.where(s > 0, s, 0.2 * s))
        o_ref[...] = jnp.where(row < mreal, e, 0.0)
    return body


def _tc_gat_exp(a, b, mreal):
    M = a.shape[0]
    return pl.pallas_call(
        _make_gat_exp_body(mreal),
        grid=(M // _BM,),
        in_specs=[
            pl.BlockSpec((_BM, CH), lambda i: (i, 0)),
            pl.BlockSpec((_BM, CH), lambda i: (i, 0)),
        ],
        out_specs=pl.BlockSpec((_BM, CH), lambda i: (i, 0)),
        out_shape=jax.ShapeDtypeStruct((M, CH), jnp.float32),
    )(a, b)


def _make_tr_logits_body(mreal):
    def body(q_ref, k_ref, e_ref, l_ref, mx_ref):
        i = pl.program_id(0)

        @pl.when(i == 0)
        def _init():
            mx_ref[...] = jnp.full_like(mx_ref, -1e30)

        l = jnp.dot(q_ref[...] * k_ref[...], e_ref[...],
                    preferred_element_type=jnp.float32)
        l_ref[...] = l
        row = i * _BM + lax.broadcasted_iota(jnp.int32, (_BM, CH), 0)
        lm = jnp.where(row < mreal, l, -1e30)
        bm = jnp.max(lm, axis=0, keepdims=True)
        mx_ref[...] = jnp.maximum(mx_ref[...],
                                  jnp.broadcast_to(bm, mx_ref.shape))
    return body


def _tc_tr_logits(q, k, expT, mreal):
    M = q.shape[0]
    return pl.pallas_call(
        _make_tr_logits_body(mreal),
        grid=(M // _BM,),
        in_specs=[
            pl.BlockSpec((_BM, 256), lambda i: (i, 0)),
            pl.BlockSpec((_BM, 256), lambda i: (i, 0)),
            pl.BlockSpec((256, CH), lambda i: (0, 0)),
        ],
        out_specs=[
            pl.BlockSpec((_BM, CH), lambda i: (i, 0)),
            pl.BlockSpec((8, CH), lambda i: (0, 0)),
        ],
        out_shape=[
            jax.ShapeDtypeStruct((M, CH), jnp.float32),
            jax.ShapeDtypeStruct((8, CH), jnp.float32),
        ],
    )(q, k, expT)


def _make_exp_body(mreal):
    def body(l_ref, m_ref, o_ref):
        i = pl.program_id(0)
        row = i * _BM + lax.broadcasted_iota(jnp.int32, (_BM, CH), 0)
        e = jnp.exp(l_ref[...] - m_ref[...])
        o_ref[...] = jnp.where(row < mreal, e, 0.0)
    return body


def _tc_exp(l, m, mreal):
    M = l.shape[0]
    return pl.pallas_call(
        _make_exp_body(mreal),
        grid=(M // _BM,),
        in_specs=[
            pl.BlockSpec((_BM, CH), lambda i: (i, 0)),
            pl.BlockSpec((1, CH), lambda i: (0, 0)),
        ],
        out_specs=pl.BlockSpec((_BM, CH), lambda i: (i, 0)),
        out_shape=jax.ShapeDtypeStruct((M, CH), jnp.float32),
    )(l, m)


def _scale_body(h_ref, n_ref, e_ref, o_ref):
    w = jnp.dot(n_ref[...], e_ref[...], preferred_element_type=jnp.float32)
    o_ref[...] = h_ref[...] * w


def _tc_scale(Hs, num, expand):
    M = Hs.shape[0]
    return pl.pallas_call(
        _scale_body,
        grid=(M // _BM,),
        in_specs=[
            pl.BlockSpec((_BM, 256), lambda i: (i, 0)),
            pl.BlockSpec((_BM, CH), lambda i: (i, 0)),
            pl.BlockSpec((CH, 256), lambda i: (0, 0)),
        ],
        out_specs=pl.BlockSpec((_BM, 256), lambda i: (i, 0)),
        out_shape=jax.ShapeDtypeStruct((M, 256), jnp.float32),
    )(Hs, num, expand)


def _stats_body(h_ref, b_ref, v_ref, s1_ref, s2_ref):
    i = pl.program_id(0)

    @pl.when(i == 0)
    def _init():
        s1_ref[...] = jnp.zeros_like(s1_ref)
        s2_ref[...] = jnp.zeros_like(s2_ref)

    v = h_ref[0] + h_ref[1] + b_ref[...]
    v_ref[...] = v
    row = i * _BM + lax.broadcasted_iota(jnp.int32, (_BM, 256), 0)
    vm = jnp.where(row < N, v, 0.0)
    s1 = jnp.sum(vm, axis=0, keepdims=True)
    s2 = jnp.sum(vm * vm, axis=0, keepdims=True)
    s1_ref[...] += jnp.broadcast_to(s1, s1_ref.shape) * 0.125
    s2_ref[...] += jnp.broadcast_to(s2, s2_ref.shape) * 0.125


def _tc_stats(halves, bias):
    return pl.pallas_call(
        _stats_body,
        grid=(NPAD // _BM,),
        in_specs=[
            pl.BlockSpec((2, _BM, 256), lambda i: (0, i, 0)),
            pl.BlockSpec((1, 256), lambda i: (0, 0)),
        ],
        out_specs=[
            pl.BlockSpec((_BM, 256), lambda i: (i, 0)),
            pl.BlockSpec((8, 256), lambda i: (0, 0)),
            pl.BlockSpec((8, 256), lambda i: (0, 0)),
        ],
        out_shape=[
            jax.ShapeDtypeStruct((NPAD, 256), jnp.float32),
            jax.ShapeDtypeStruct((8, 256), jnp.float32),
            jax.ShapeDtypeStruct((8, 256), jnp.float32),
        ],
    )(halves, bias.reshape(1, 256))


def _affine_body(v_ref, s_ref, t_ref, o_ref):
    o_ref[...] = jnp.maximum(v_ref[...] * s_ref[...] + t_ref[...], 0.0)


def _affine_res_body(v_ref, s_ref, t_ref, r_ref, o_ref):
    o_ref[...] = jnp.maximum(
        v_ref[...] * s_ref[...] + t_ref[...], 0.0) + r_ref[...]


def _tc_affine(v, scale, shift, res=None):
    ins = [v, scale.reshape(1, 256), shift.reshape(1, 256)]
    specs = [
        pl.BlockSpec((_BM, 256), lambda i: (i, 0)),
        pl.BlockSpec((1, 256), lambda i: (0, 0)),
        pl.BlockSpec((1, 256), lambda i: (0, 0)),
    ]
    body = _affine_body
    if res is not None:
        ins.append(res)
        specs.append(pl.BlockSpec((_BM, 256), lambda i: (i, 0)))
        body = _affine_res_body
    return pl.pallas_call(
        body,
        grid=(NPAD // _BM,),
        in_specs=specs,
        out_specs=pl.BlockSpec((_BM, 256), lambda i: (i, 0)),
        out_shape=jax.ShapeDtypeStruct((NPAD, 256), jnp.float32),
    )(*ins)


def _norm_relu_body(h_ref, e_ref, b_ref, o_ref):
    u = h_ref[0, :, :256] + h_ref[1, :, :256]
    s = h_ref[0, :, 256:] + h_ref[1, :, 256:]
    den = jnp.dot(s, e_ref[...], preferred_element_type=jnp.float32)
    o_ref[...] = jnp.maximum(u / (den + 1e-16) + b_ref[...], 0.0)


def _tc_norm_relu(halves, expand, bias):
    return pl.pallas_call(
        _norm_relu_body,
        grid=(NPAD // _BM,),
        in_specs=[
            pl.BlockSpec((2, _BM, 384), lambda i: (0, i, 0)),
            pl.BlockSpec((CH, 256), lambda i: (0, 0)),
            pl.BlockSpec((1, 256), lambda i: (0, 0)),
        ],
        out_specs=pl.BlockSpec((_BM, 256), lambda i: (i, 0)),
        out_shape=jax.ShapeDtypeStruct((NPAD, 256), jnp.float32),
    )(halves, expand, bias.reshape(1, 256))


def _pool_body(g_ref, a_ref, w_ref, t_ref, e_ref, b_ref, sum_ref, cnt_ref):
    i = pl.program_id(0)

    @pl.when(i == 0)
    def _init():
        sum_ref[...] = jnp.zeros_like(sum_ref)
        cnt_ref[...] = jnp.zeros_like(cnt_ref)

    u = t_ref[0, :, :256] + t_ref[1, :, :256]
    s = t_ref[0, :, 256:] + t_ref[1, :, 256:]
    den = jnp.dot(s, e_ref[...], preferred_element_type=jnp.float32)
    c = g_ref[...] + a_ref[...] + w_ref[...] + u / (den + 1e-16)
    row = i * _BM + lax.broadcasted_iota(jnp.int32, (_BM, 256), 0)
    c = jnp.where(row < N, c, 0.0)
    b = b_ref[0, 0, :]
    gids = lax.broadcasted_iota(jnp.int32, (G, _BM), 0)
    onehot = (b[None, :] == gids).astype(jnp.float32)
    sum_ref[...] += jnp.dot(onehot, c, preferred_element_type=jnp.float32)
    cnt_ref[...] += jnp.sum(onehot, axis=1, keepdims=True)


def _tc_pool(gcn, gat, tws, thalves, expand, batch_p):
    grid = NPAD // _BM
    sums, cnt = pl.pallas_call(
        _pool_body,
        grid=(grid,),
        in_specs=[
            pl.BlockSpec((_BM, 256), lambda i: (i, 0)),
            pl.BlockSpec((_BM, 256), lambda i: (i, 0)),
            pl.BlockSpec((_BM, 256), lambda i: (i, 0)),
            pl.BlockSpec((2, _BM, 384), lambda i: (0, i, 0)),
            pl.BlockSpec((CH, 256), lambda i: (0, 0)),
            pl.BlockSpec((1, 1, _BM), lambda i: (i, 0, 0)),
        ],
        out_specs=[
            pl.BlockSpec((G, 256), lambda i: (0, 0)),
            pl.BlockSpec((G, 1), lambda i: (0, 0)),
        ],
        out_shape=[
            jax.ShapeDtypeStruct((G, 256), jnp.float32),
            jax.ShapeDtypeStruct((G, 1), jnp.float32),
        ],
    )(gcn, gat, tws, thalves, expand, batch_p.reshape(grid, 1, _BM))
    return sums / jnp.maximum(cnt, 1.0)


def _head_expand(heads, oc):
    col = jnp.arange(heads * oc) // oc
    return (col[None, :] == jnp.arange(CH)[:, None]).astype(jnp.float32)


def kernel(x, edge_index, batch, params):
    p = params
    i32 = jnp.int32
    src = edge_index[0].astype(i32)
    dst = edge_index[1].astype(i32)
    loop = jnp.arange(N, dtype=i32)
    s_up = jnp.concatenate([src, loop, jnp.zeros((EP - EUP,), i32)])
    d_up = jnp.concatenate([dst, loop, jnp.full((EP - EUP,), N, i32)])
    s_t = jnp.concatenate([src, jnp.zeros((EPT - E,), i32)])
    d_t = jnp.concatenate([dst, jnp.full((EPT - E,), N, i32)])
    xp = jnp.zeros((NPAD, IN_DIM), jnp.float32).at[:N].set(
        x.astype(jnp.float32))
    batch_p = jnp.concatenate(
        [batch.astype(i32), jnp.full((NPAD - N,), 127, i32)])

    z256 = jnp.zeros((256,), jnp.float32)
    zch = jnp.zeros((CH,), jnp.float32)

    degh = _sc_degree(d_up)
    dinv_tab = _tc_dinv(degh)
    ds_g = _sc_gather(dinv_tab, s_up, CH)
    dd_g = _sc_gather(dinv_tab, d_up, CH)
    coeff = _tc_mul(ds_g, dd_g, EUP)
    expand_gcn = _head_expand(1, 256)

    gcn_out = xp
    for i in range(4):
        h = _tc_matmul(gcn_out, p['gcn_W%d' % i], z256)
        Hs = _sc_gather(h, s_up, 256)
        msg = _tc_scale(Hs, coeff, expand_gcn)
        halves = _sc_segment_sum([msg], d_up)
        v, s1, s2 = _tc_stats(halves, p['gcn_b%d' % i])
        mu = s1.sum(0) / N
        var = s2.sum(0) / N - mu * mu
        sc = p['bn_g%d' % i] * lax.rsqrt(var + 1e-5)
        sh = p['bn_b%d' % i] - mu * sc
        res = gcn_out if i > 0 else None
        gcn_out = _tc_affine(v, sc, sh, res)

    gat_out = xp
    for i, (hds, oc) in enumerate([(8, 32), (4, 64), (1, 256)]):
        h = _tc_matmul(gat_out, p['gat%d_W' % i], z256)
        hcol = jnp.repeat(jnp.arange(hds), oc)
        As = jnp.zeros((256, CH), jnp.float32).at[
            jnp.arange(256), hcol].set(p['gat%d_as' % i].reshape(-1))
        Ad = jnp.zeros((256, CH), jnp.float32).at[
            jnp.arange(256), hcol].set(p['gat%d_ad' % i].reshape(-1))
        als = _tc_matmul(h, As, zch)
        ald = _tc_matmul(h, Ad, zch)
        as_g = _sc_gather(als, s_up, CH)
        ad_g = _sc_gather(ald, d_up, CH)
        eexp = _tc_gat_exp(as_g, ad_g, EUP)
        Hs = _sc_gather(h, s_up, 256)
        expand_i = _head_expand(hds, oc)
        msg = _tc_scale(Hs, eexp, expand_i)
        halves = _sc_segment_sum([msg, eexp], d_up)
        gat_out = _tc_norm_relu(halves, expand_i, p['gat%d_b' % i])

    q = _tc_matmul(gcn_out, p['tr_Wq'], z256)
    kk = _tc_matmul(gcn_out, p['tr_Wk'], z256)
    vv = _tc_matmul(gcn_out, p['tr_Wv'], z256)
    tws = _tc_matmul(gcn_out, p['tr_Ws'], p['tr_b'])
    Qd = _sc_gather(q, d_t, 256)
    Ks = _sc_gather(kk, s_t, 256)
    expand8 = _head_expand(8, 32)
    expT = expand8.T * (1.0 / jnp.sqrt(32.0))
    l, mx = _tc_tr_logits(Qd, Ks, expT, E)
    m = jnp.full((1, CH), jnp.max(mx))
    eexp = _tc_exp(l, m, E)
    Vs = _sc_gather(vv, s_t, 256)
    msg = _tc_scale(Vs, eexp, expand8)
    thalves = _sc_segment_sum([msg, eexp], d_t)

    return _tc_pool(gcn_out, gat_out, tws, thalves, expand8, batch_p)


# speedup vs baseline: 14.0923x; 1.3833x over previous
import jax
import jax.numpy as jnp
from jax import lax
from jax.experimental import pallas as pl
from jax.experimental.pallas import tpu as pltpu
from jax.experimental.pallas import tpu_sc as plsc

N = 10000
E = 320000
G = 8
IN_DIM = 128
HID = 256
OUT = 256

NPAD = 10240
NC, NS = 2, 16
NW = NC * NS
EB = 128
EUP = E + N
EP = ((EUP + NW * EB - 1) // (NW * EB)) * (NW * EB)
EPT = ((E + NW * EB - 1) // (NW * EB)) * (NW * EB)
CH = 128

_mesh = plsc.VectorSubcoreMesh(
    core_axis_name="c", subcore_axis_name="s", num_cores=NC, num_subcores=NS)


def _sc_gather(table, idx, D):
    M = idx.shape[0]
    perw = M // NW
    nb = perw // EB

    def body(table_hbm, idx_hbm, out_hbm, idx_v, rows_v, sem):
        wid = lax.axis_index("s") * NC + lax.axis_index("c")
        base = wid * perw

        def step(j, c):
            off = base + j * EB
            pltpu.sync_copy(idx_hbm.at[pl.ds(off, EB)], idx_v)
            pltpu.async_copy(table_hbm.at[idx_v], rows_v, sem).wait()
            pltpu.sync_copy(rows_v, out_hbm.at[pl.ds(off, EB)])
            return c

        lax.fori_loop(0, nb, step, 0)

    f = pl.kernel(
        body,
        out_type=jax.ShapeDtypeStruct((M, D), jnp.float32),
        mesh=_mesh,
        scratch_types=[
            pltpu.VMEM((EB,), jnp.int32),
            pltpu.VMEM((EB, D), jnp.float32),
            pltpu.SemaphoreType.DMA,
        ],
    )
    return f(table, idx)


def _sc_segment_sum(parts, idx):
    M = idx.shape[0]
    perw = M // NW
    nb = perw // EB
    chunks = []
    for pi, a in enumerate(parts):
        for off in range(0, a.shape[1], CH):
            chunks.append((pi, off))
    Dtot = CH * len(chunks)
    RS = NPAD // NS

    def body(*refs):
        part_hbm = refs[:len(parts)]
        idx_hbm, out_hbm, idx_v, buf, acc = refs[len(parts):]
        cid = lax.axis_index("c")
        sid = lax.axis_index("s")
        base = (sid * NC + cid) * perw
        zero16 = jnp.zeros((16,), jnp.float32)

        for ci, (pi, off) in enumerate(chunks):
            def zrow(r, c):
                for jj in range(CH // 16):
                    buf[r, pl.ds(jj * 16, 16)] = zero16
                return c

            lax.fori_loop(0, EB, zrow, 0)
            for t in range(RS // EB):
                pltpu.sync_copy(buf, acc.at[pl.ds(sid * RS + t * EB, EB)])
            plsc.subcore_barrier()

            def step(j, c):
                eoff = base + j * EB
                pltpu.sync_copy(idx_hbm.at[pl.ds(eoff, EB)], idx_v)
                pltpu.sync_copy(
                    part_hbm[pi].at[pl.ds(eoff, EB), pl.ds(off, CH)], buf)
                pltpu.sync_copy(buf, acc.at[idx_v], add=True)
                return c

            lax.fori_loop(0, nb, step, 0)
            plsc.subcore_barrier()
            for t in range(RS // EB):
                r0 = sid * RS + t * EB
                pltpu.sync_copy(
                    acc.at[pl.ds(r0, EB)],
                    out_hbm.at[cid, pl.ds(r0, EB), pl.ds(ci * CH, CH)])
            plsc.subcore_barrier()

    f = pl.kernel(
        body,
        out_type=jax.ShapeDtypeStruct((NC, NPAD, Dtot), jnp.float32),
        mesh=_mesh,
        scratch_types=[
            pltpu.VMEM((EB,), jnp.int32),
            pltpu.VMEM((EB, CH), jnp.float32),
            pltpu.VMEM_SHARED((NPAD, CH), jnp.float32),
        ],
    )
    return f(*parts, idx)


def _sc_gather_scatter(table, sidx, didx):
    M = sidx.shape[0]
    perw = M // NW
    nb = perw // EB
    RS = NPAD // NS

    def body(tab_hbm, sidx_hbm, didx_hbm, out_hbm, si_v, di_v, buf, acc, sem):
        cid = lax.axis_index("c")
        sid = lax.axis_index("s")
        base = (sid * NC + cid) * perw
        zero16 = jnp.zeros((16,), jnp.float32)

        for ch in range(2):
            def zrow(r, c):
                for jj in range(CH // 16):
                    buf[r, pl.ds(jj * 16, 16)] = zero16
                return c

            lax.fori_loop(0, EB, zrow, 0)
            for t in range(RS // EB):
                pltpu.sync_copy(buf, acc.at[pl.ds(sid * RS + t * EB, EB)])
            plsc.subcore_barrier()

            def step(j, c):
                eoff = base + j * EB
                pltpu.sync_copy(sidx_hbm.at[pl.ds(eoff, EB)], si_v)
                pltpu.sync_copy(didx_hbm.at[pl.ds(eoff, EB)], di_v)
                pltpu.async_copy(
                    tab_hbm.at[si_v, pl.ds(ch * CH, CH)], buf, sem).wait()
                pltpu.sync_copy(buf, acc.at[di_v], add=True)
                return c

            lax.fori_loop(0, nb, step, 0)
            plsc.subcore_barrier()
            for t in range(RS // EB):
                r0 = sid * RS + t * EB
                pltpu.sync_copy(
                    acc.at[pl.ds(r0, EB)],
                    out_hbm.at[cid, pl.ds(r0, EB), pl.ds(ch * CH, CH)])
            plsc.subcore_barrier()

    f = pl.kernel(
        body,
        out_type=jax.ShapeDtypeStruct((NC, NPAD, 256), jnp.float32),
        mesh=_mesh,
        scratch_types=[
            pltpu.VMEM((EB,), jnp.int32),
            pltpu.VMEM((EB,), jnp.int32),
            pltpu.VMEM((EB, CH), jnp.float32),
            pltpu.VMEM_SHARED((NPAD, CH), jnp.float32),
            pltpu.SemaphoreType.DMA,
        ],
    )
    return f(table, sidx, didx)


def _sc_degree(idx):
    M = idx.shape[0]
    perw = M // NW
    nb = perw // EB
    RS = NPAD // NS

    def body(idx_hbm, out_hbm, idx_v, buf, acc):
        cid = lax.axis_index("c")
        sid = lax.axis_index("s")
        base = (sid * NC + cid) * perw
        zero16 = jnp.zeros((16,), jnp.float32)

        def zrow(r, c):
            for jj in range(CH // 16):
                buf[r, pl.ds(jj * 16, 16)] = zero16
            return c

        lax.fori_loop(0, EB, zrow, 0)
        for t in range(RS // EB):
            pltpu.sync_copy(buf, acc.at[pl.ds(sid * RS + t * EB, EB)])
        plsc.subcore_barrier()

        one16 = jnp.ones((16,), jnp.float32)

        def orow(r, c):
            for jj in range(CH // 16):
                buf[r, pl.ds(jj * 16, 16)] = one16
            return c

        lax.fori_loop(0, EB, orow, 0)

        def step(j, c):
            eoff = base + j * EB
            pltpu.sync_copy(idx_hbm.at[pl.ds(eoff, EB)], idx_v)
            pltpu.sync_copy(buf, acc.at[idx_v], add=True)
            return c

        lax.fori_loop(0, nb, step, 0)
        plsc.subcore_barrier()
        for t in range(RS // EB):
            r0 = sid * RS + t * EB
            pltpu.sync_copy(acc.at[pl.ds(r0, EB)],
                            out_hbm.at[cid, pl.ds(r0, EB)])
        plsc.subcore_barrier()

    f = pl.kernel(
        body,
        out_type=jax.ShapeDtypeStruct((NC, NPAD, CH), jnp.float32),
        mesh=_mesh,
        scratch_types=[
            pltpu.VMEM((EB,), jnp.int32),
            pltpu.VMEM((EB, CH), jnp.float32),
            pltpu.VMEM_SHARED((NPAD, CH), jnp.float32),
        ],
    )
    return f(idx)


_BM = 512


def _mm_body(x_ref, w_ref, b_ref, o_ref):
    o_ref[...] = jnp.dot(
        x_ref[...], w_ref[...], preferred_element_type=jnp.float32
    ) + b_ref[...]


def _tc_matmul(x, W, bias):
    M, K = x.shape
    Dout = W.shape[1]
    return pl.pallas_call(
        _mm_body,
        grid=(M // _BM,),
        in_specs=[
            pl.BlockSpec((_BM, K), lambda i: (i, 0)),
            pl.BlockSpec((K, Dout), lambda i: (0, 0)),
            pl.BlockSpec((1, Dout), lambda i: (0, 0)),
        ],
        out_specs=pl.BlockSpec((_BM, Dout), lambda i: (i, 0)),
        out_shape=jax.ShapeDtypeStruct((M, Dout), jnp.float32),
    )(x, W, bias.reshape(1, Dout))


def _mm_scaled_body(x_ref, w_ref, d_ref, o_ref):
    o_ref[...] = jnp.dot(
        x_ref[...], w_ref[...], preferred_element_type=jnp.float32
    ) * d_ref[...][:, 0:1]


def _tc_matmul_rowscale(x, W, dinv_tab):
    M, K = x.shape
    Dout = W.shape[1]
    return pl.pallas_call(
        _mm_scaled_body,
        grid=(M // _BM,),
        in_specs=[
            pl.BlockSpec((_BM, K), lambda i: (i, 0)),
            pl.BlockSpec((K, Dout), lambda i: (0, 0)),
            pl.BlockSpec((_BM, CH), lambda i: (i, 0)),
        ],
        out_specs=pl.BlockSpec((_BM, Dout), lambda i: (i, 0)),
        out_shape=jax.ShapeDtypeStruct((M, Dout), jnp.float32),
    )(x, W, dinv_tab)


def _dinv_body(h_ref, o_ref):
    d = h_ref[0] + h_ref[1]
    o_ref[...] = jnp.where(d > 0, lax.rsqrt(jnp.maximum(d, 1e-12)), 0.0)


def _tc_dinv(halves):
    return pl.pallas_call(
        _dinv_body,
        grid=(NPAD // _BM,),
        in_specs=[pl.BlockSpec((2, _BM, CH), lambda i: (0, i, 0))],
        out_specs=pl.BlockSpec((_BM, CH), lambda i: (i, 0)),
        out_shape=jax.ShapeDtypeStruct((NPAD, CH), jnp.float32),
    )(halves)


def _make_mul_body(mreal):
    def body(a_ref, b_ref, o_ref):
        i = pl.program_id(0)
        row = i * _BM + lax.broadcasted_iota(jnp.int32, (_BM, CH), 0)
        o_ref[...] = jnp.where(row < mreal, a_ref[...] * b_ref[...], 0.0)
    return body


def _tc_mul(a, b, mreal):
    M, C = a.shape
    return pl.pallas_call(
        _make_mul_body(mreal),
        grid=(M // _BM,),
        in_specs=[
            pl.BlockSpec((_BM, C), lambda i: (i, 0)),
            pl.BlockSpec((_BM, C), lambda i: (i, 0)),
        ],
        out_specs=pl.BlockSpec((_BM, C), lambda i: (i, 0)),
        out_shape=jax.ShapeDtypeStruct((M, C), jnp.float32),
    )(a, b)


def _make_gat_exp_body(mreal):
    def body(a_ref, b_ref, o_ref):
        i = pl.program_id(0)
        row = i * _BM + lax.broadcasted_iota(jnp.int32, (_BM, CH), 0)
        s = a_ref[...] + b_ref[...]
        e = jnp.exp(jnp.where(s > 0, s, 0.2 * s))
        o_ref[...] = jnp.where(row < mreal, e, 0.0)
    return body


def _tc_gat_exp(a, b, mreal):
    M = a.shape[0]
    return pl.pallas_call(
        _make_gat_exp_body(mreal),
        grid=(M // _BM,),
        in_specs=[
            pl.BlockSpec((_BM, CH), lambda i: (i, 0)),
            pl.BlockSpec((_BM, CH), lambda i: (i, 0)),
        ],
        out_specs=pl.BlockSpec((_BM, CH), lambda i: (i, 0)),
        out_shape=jax.ShapeDtypeStruct((M, CH), jnp.float32),
    )(a, b)


def _make_tr_logits_body(mreal):
    def body(q_ref, k_ref, e_ref, l_ref, mx_ref):
        i = pl.program_id(0)

        @pl.when(i == 0)
        def _init():
            mx_ref[...] = jnp.full_like(mx_ref, -1e30)

        l = jnp.dot(q_ref[...] * k_ref[...], e_ref[...],
                    preferred_element_type=jnp.float32)
        l_ref[...] = l
        row = i * _BM + lax.broadcasted_iota(jnp.int32, (_BM, CH), 0)
        lm = jnp.where(row < mreal, l, -1e30)
        bm = jnp.max(lm, axis=0, keepdims=True)
        mx_ref[...] = jnp.maximum(mx_ref[...],
                                  jnp.broadcast_to(bm, mx_ref.shape))
    return body


def _tc_tr_logits(q, k, expT, mreal):
    M = q.shape[0]
    return pl.pallas_call(
        _make_tr_logits_body(mreal),
        grid=(M // _BM,),
        in_specs=[
            pl.BlockSpec((_BM, 256), lambda i: (i, 0)),
            pl.BlockSpec((_BM, 256), lambda i: (i, 0)),
            pl.BlockSpec((256, CH), lambda i: (0, 0)),
        ],
        out_specs=[
            pl.BlockSpec((_BM, CH), lambda i: (i, 0)),
            pl.BlockSpec((8, CH), lambda i: (0, 0)),
        ],
        out_shape=[
            jax.ShapeDtypeStruct((M, CH), jnp.float32),
            jax.ShapeDtypeStruct((8, CH), jnp.float32),
        ],
    )(q, k, expT)


def _make_exp_body(mreal):
    def body(l_ref, m_ref, o_ref):
        i = pl.program_id(0)
        row = i * _BM + lax.broadcasted_iota(jnp.int32, (_BM, CH), 0)
        e = jnp.exp(l_ref[...] - m_ref[...])
        o_ref[...] = jnp.where(row < mreal, e, 0.0)
    return body


def _tc_exp(l, m, mreal):
    M = l.shape[0]
    return pl.pallas_call(
        _make_exp_body(mreal),
        grid=(M // _BM,),
        in_specs=[
            pl.BlockSpec((_BM, CH), lambda i: (i, 0)),
            pl.BlockSpec((1, CH), lambda i: (0, 0)),
        ],
        out_specs=pl.BlockSpec((_BM, CH), lambda i: (i, 0)),
        out_shape=jax.ShapeDtypeStruct((M, CH), jnp.float32),
    )(l, m)


def _scale_body(h_ref, n_ref, e_ref, o_ref):
    w = jnp.dot(n_ref[...], e_ref[...], preferred_element_type=jnp.float32)
    o_ref[...] = h_ref[...] * w


def _tc_scale(Hs, num, expand):
    M = Hs.shape[0]
    return pl.pallas_call(
        _scale_body,
        grid=(M // _BM,),
        in_specs=[
            pl.BlockSpec((_BM, 256), lambda i: (i, 0)),
            pl.BlockSpec((_BM, CH), lambda i: (i, 0)),
            pl.BlockSpec((CH, 256), lambda i: (0, 0)),
        ],
        out_specs=pl.BlockSpec((_BM, 256), lambda i: (i, 0)),
        out_shape=jax.ShapeDtypeStruct((M, 256), jnp.float32),
    )(Hs, num, expand)


def _stats_body(h_ref, b_ref, d_ref, v_ref, s1_ref, s2_ref):
    i = pl.program_id(0)

    @pl.when(i == 0)
    def _init():
        s1_ref[...] = jnp.zeros_like(s1_ref)
        s2_ref[...] = jnp.zeros_like(s2_ref)

    v = (h_ref[0] + h_ref[1]) * d_ref[...][:, 0:1] + b_ref[...]
    v_ref[...] = v
    row = i * _BM + lax.broadcasted_iota(jnp.int32, (_BM, 256), 0)
    vm = jnp.where(row < N, v, 0.0)
    s1 = jnp.sum(vm, axis=0, keepdims=True)
    s2 = jnp.sum(vm * vm, axis=0, keepdims=True)
    s1_ref[...] += jnp.broadcast_to(s1, s1_ref.shape) * 0.125
    s2_ref[...] += jnp.broadcast_to(s2, s2_ref.shape) * 0.125


def _tc_stats(halves, bias, dinv_tab):
    return pl.pallas_call(
        _stats_body,
        grid=(NPAD // _BM,),
        in_specs=[
            pl.BlockSpec((2, _BM, 256), lambda i: (0, i, 0)),
            pl.BlockSpec((1, 256), lambda i: (0, 0)),
            pl.BlockSpec((_BM, CH), lambda i: (i, 0)),
        ],
        out_specs=[
            pl.BlockSpec((_BM, 256), lambda i: (i, 0)),
            pl.BlockSpec((8, 256), lambda i: (0, 0)),
            pl.BlockSpec((8, 256), lambda i: (0, 0)),
        ],
        out_shape=[
            jax.ShapeDtypeStruct((NPAD, 256), jnp.float32),
            jax.ShapeDtypeStruct((8, 256), jnp.float32),
            jax.ShapeDtypeStruct((8, 256), jnp.float32),
        ],
    )(halves, bias.reshape(1, 256), dinv_tab)


def _make_gat_msg_body(mreal):
    def body(hs_ref, hd_ref, as_ref, ad_ref, ex_ref, m_ref, e_ref):
        i = pl.program_id(0)
        hs = hs_ref[...]
        a = jnp.dot(hs, as_ref[...], preferred_element_type=jnp.float32)
        a += jnp.dot(hd_ref[...], ad_ref[...],
                     preferred_element_type=jnp.float32)
        lk = jnp.where(a > 0, a, 0.2 * a)
        row = i * _BM + lax.broadcasted_iota(jnp.int32, (_BM, CH), 0)
        e = jnp.where(row < mreal, jnp.exp(lk), 0.0)
        e_ref[...] = e
        w = jnp.dot(e, ex_ref[...], preferred_element_type=jnp.float32)
        m_ref[...] = hs * w
    return body


def _tc_gat_msg(Hs, Hd, As, Ad, expand, mreal):
    M = Hs.shape[0]
    return pl.pallas_call(
        _make_gat_msg_body(mreal),
        grid=(M // _BM,),
        in_specs=[
            pl.BlockSpec((_BM, 256), lambda i: (i, 0)),
            pl.BlockSpec((_BM, 256), lambda i: (i, 0)),
            pl.BlockSpec((256, CH), lambda i: (0, 0)),
            pl.BlockSpec((256, CH), lambda i: (0, 0)),
            pl.BlockSpec((CH, 256), lambda i: (0, 0)),
        ],
        out_specs=[
            pl.BlockSpec((_BM, 256), lambda i: (i, 0)),
            pl.BlockSpec((_BM, CH), lambda i: (i, 0)),
        ],
        out_shape=[
            jax.ShapeDtypeStruct((M, 256), jnp.float32),
            jax.ShapeDtypeStruct((M, CH), jnp.float32),
        ],
    )(Hs, Hd, As, Ad, expand)


def _affine_body(v_ref, s_ref, t_ref, o_ref):
    o_ref[...] = jnp.maximum(v_ref[...] * s_ref[...] + t_ref[...], 0.0)


def _affine_res_body(v_ref, s_ref, t_ref, r_ref, o_ref):
    o_ref[...] = jnp.maximum(
        v_ref[...] * s_ref[...] + t_ref[...], 0.0) + r_ref[...]


def _tc_affine(v, scale, shift, res=None):
    ins = [v, scale.reshape(1, 256), shift.reshape(1, 256)]
    specs = [
        pl.BlockSpec((_BM, 256), lambda i: (i, 0)),
        pl.BlockSpec((1, 256), lambda i: (0, 0)),
        pl.BlockSpec((1, 256), lambda i: (0, 0)),
    ]
    body = _affine_body
    if res is not None:
        ins.append(res)
        specs.append(pl.BlockSpec((_BM, 256), lambda i: (i, 0)))
        body = _affine_res_body
    return pl.pallas_call(
        body,
        grid=(NPAD // _BM,),
        in_specs=specs,
        out_specs=pl.BlockSpec((_BM, 256), lambda i: (i, 0)),
        out_shape=jax.ShapeDtypeStruct((NPAD, 256), jnp.float32),
    )(*ins)


def _norm_relu_body(h_ref, e_ref, b_ref, o_ref):
    u = h_ref[0, :, :256] + h_ref[1, :, :256]
    s = h_ref[0, :, 256:] + h_ref[1, :, 256:]
    den = jnp.dot(s, e_ref[...], preferred_element_type=jnp.float32)
    o_ref[...] = jnp.maximum(u / (den + 1e-16) + b_ref[...], 0.0)


def _tc_norm_relu(halves, expand, bias):
    return pl.pallas_call(
        _norm_relu_body,
        grid=(NPAD // _BM,),
        in_specs=[
            pl.BlockSpec((2, _BM, 384), lambda i: (0, i, 0)),
            pl.BlockSpec((CH, 256), lambda i: (0, 0)),
            pl.BlockSpec((1, 256), lambda i: (0, 0)),
        ],
        out_specs=pl.BlockSpec((_BM, 256), lambda i: (i, 0)),
        out_shape=jax.ShapeDtypeStruct((NPAD, 256), jnp.float32),
    )(halves, expand, bias.reshape(1, 256))


def _pool_body(g_ref, a_ref, w_ref, t_ref, e_ref, b_ref, sum_ref, cnt_ref):
    i = pl.program_id(0)

    @pl.when(i == 0)
    def _init():
        sum_ref[...] = jnp.zeros_like(sum_ref)
        cnt_ref[...] = jnp.zeros_like(cnt_ref)

    u = t_ref[0, :, :256] + t_ref[1, :, :256]
    s = t_ref[0, :, 256:] + t_ref[1, :, 256:]
    den = jnp.dot(s, e_ref[...], preferred_element_type=jnp.float32)
    c = g_ref[...] + a_ref[...] + w_ref[...] + u / (den + 1e-16)
    row = i * _BM + lax.broadcasted_iota(jnp.int32, (_BM, 256), 0)
    c = jnp.where(row < N, c, 0.0)
    b = b_ref[0, 0, :]
    gids = lax.broadcasted_iota(jnp.int32, (G, _BM), 0)
    onehot = (b[None, :] == gids).astype(jnp.float32)
    sum_ref[...] += jnp.dot(onehot, c, preferred_element_type=jnp.float32)
    cnt_ref[...] += jnp.sum(onehot, axis=1, keepdims=True)


def _tc_pool(gcn, gat, tws, thalves, expand, batch_p):
    grid = NPAD // _BM
    sums, cnt = pl.pallas_call(
        _pool_body,
        grid=(grid,),
        in_specs=[
            pl.BlockSpec((_BM, 256), lambda i: (i, 0)),
            pl.BlockSpec((_BM, 256), lambda i: (i, 0)),
            pl.BlockSpec((_BM, 256), lambda i: (i, 0)),
            pl.BlockSpec((2, _BM, 384), lambda i: (0, i, 0)),
            pl.BlockSpec((CH, 256), lambda i: (0, 0)),
            pl.BlockSpec((1, 1, _BM), lambda i: (i, 0, 0)),
        ],
        out_specs=[
            pl.BlockSpec((G, 256), lambda i: (0, 0)),
            pl.BlockSpec((G, 1), lambda i: (0, 0)),
        ],
        out_shape=[
            jax.ShapeDtypeStruct((G, 256), jnp.float32),
            jax.ShapeDtypeStruct((G, 1), jnp.float32),
        ],
    )(gcn, gat, tws, thalves, expand, batch_p.reshape(grid, 1, _BM))
    return sums / jnp.maximum(cnt, 1.0)


def _head_expand(heads, oc):
    col = jnp.arange(heads * oc) // oc
    return (col[None, :] == jnp.arange(CH)[:, None]).astype(jnp.float32)


def kernel(x, edge_index, batch, params):
    p = params
    i32 = jnp.int32
    src = edge_index[0].astype(i32)
    dst = edge_index[1].astype(i32)
    loop = jnp.arange(N, dtype=i32)
    s_up = jnp.concatenate([src, loop, jnp.zeros((EP - EUP,), i32)])
    d_up = jnp.concatenate([dst, loop, jnp.full((EP - EUP,), N, i32)])
    s_t = jnp.concatenate([src, jnp.zeros((EPT - E,), i32)])
    d_t = jnp.concatenate([dst, jnp.full((EPT - E,), N, i32)])
    xp = jnp.zeros((NPAD, IN_DIM), jnp.float32).at[:N].set(
        x.astype(jnp.float32))
    batch_p = jnp.concatenate(
        [batch.astype(i32), jnp.full((NPAD - N,), 127, i32)])

    z256 = jnp.zeros((256,), jnp.float32)
    zch = jnp.zeros((CH,), jnp.float32)

    degh = _sc_degree(d_up)
    dinv_tab = _tc_dinv(degh)

    gcn_out = xp
    for i in range(4):
        h = _tc_matmul_rowscale(gcn_out, p['gcn_W%d' % i], dinv_tab)
        halves = _sc_gather_scatter(h, s_up, d_up)
        v, s1, s2 = _tc_stats(halves, p['gcn_b%d' % i], dinv_tab)
        mu = s1.sum(0) / N
        var = s2.sum(0) / N - mu * mu
        sc = p['bn_g%d' % i] * lax.rsqrt(var + 1e-5)
        sh = p['bn_b%d' % i] - mu * sc
        res = gcn_out if i > 0 else None
        gcn_out = _tc_affine(v, sc, sh, res)

    gat_out = xp
    for i, (hds, oc) in enumerate([(8, 32), (4, 64), (1, 256)]):
        h = _tc_matmul(gat_out, p['gat%d_W' % i], z256)
        hcol = jnp.repeat(jnp.arange(hds), oc)
        As = jnp.zeros((256, CH), jnp.float32).at[
            jnp.arange(256), hcol].set(p['gat%d_as' % i].reshape(-1))
        Ad = jnp.zeros((256, CH), jnp.float32).at[
            jnp.arange(256), hcol].set(p['gat%d_ad' % i].reshape(-1))
        Hs = _sc_gather(h, s_up, 256)
        Hd = _sc_gather(h, d_up, 256)
        expand_i = _head_expand(hds, oc)
        msg, eexp = _tc_gat_msg(Hs, Hd, As, Ad, expand_i, EUP)
        halves = _sc_segment_sum([msg, eexp], d_up)
        gat_out = _tc_norm_relu(halves, expand_i, p['gat%d_b' % i])

    q = _tc_matmul(gcn_out, p['tr_Wq'], z256)
    kk = _tc_matmul(gcn_out, p['tr_Wk'], z256)
    vv = _tc_matmul(gcn_out, p['tr_Wv'], z256)
    tws = _tc_matmul(gcn_out, p['tr_Ws'], p['tr_b'])
    Qd = _sc_gather(q, d_t, 256)
    Ks = _sc_gather(kk, s_t, 256)
    expand8 = _head_expand(8, 32)
    expT = expand8.T * (1.0 / jnp.sqrt(32.0))
    l, mx = _tc_tr_logits(Qd, Ks, expT, E)
    m = jnp.full((1, CH), jnp.max(mx))
    eexp = _tc_exp(l, m, E)
    Vs = _sc_gather(vv, s_t, 256)
    msg = _tc_scale(Vs, eexp, expand8)
    thalves = _sc_segment_sum([msg, eexp], d_t)

    return _tc_pool(gcn_out, gat_out, tws, thalves, expand8, batch_p)
